# baseline (speedup 1.0000x reference)
import functools

import jax
import jax.numpy as jnp
import numpy as np
from jax import lax
from jax.experimental import pallas as pl
from jax.experimental.pallas import tpu as pltpu
from jax.experimental.pallas import tpu_sc as plsc

VOCAB = 100000
D0, D1, D2 = 64, 96, 128
AGG = 128
BATCH = 4096

_R0 = VOCAB * D0 // (2 * AGG)
_R1 = VOCAB * D1 // (2 * AGG)
_W0W = D0 // 2
_W1W = D1 // 2

_info = plsc.get_sparse_core_info()
_NC, _NS, _L = _info.num_cores, _info.num_subcores, _info.num_lanes
_NW = _NC * _NS
_BPW = BATCH // _NW

_THIRD = float(np.float32(1.0) / np.float32(3.0))
_HI_MASK = np.int32(-65536)


def _interleave_perm(d):
    perm = np.empty((d,), np.int32)
    for g in range(0, d, 2 * _L):
        for k in range(_L):
            perm[g + 2 * k] = g + k
            perm[g + 2 * k + 1] = g + _L + k
    return perm

_PERM0 = _interleave_perm(D0)
_PERM1 = _interleave_perm(D1)


def _mean_kernel(idx_hbm, w0_hbm, w1_hbm, w2_hbm, out_hbm,
                 idx_v, q0_v, q1a_v, q1b_v, r0, r1a, r1b, r2, sem):
    wid = lax.axis_index("s") * _NC + lax.axis_index("c")
    base = wid * _BPW

    pltpu.sync_copy(idx_hbm.at[pl.ds(base, _BPW)], idx_v)
    for k in range(_BPW // _L):
        sl = pl.ds(k * _L, _L)
        vec = idx_v[sl]
        q0_v[sl] = lax.shift_right_logical(vec, 2)
        t = lax.shift_right_logical(vec * 3, 3)
        q1a_v[sl] = t
        q1b_v[sl] = lax.min(t + 1, jnp.full((_L,), _R1 - 1, jnp.int32))

    c0 = pltpu.async_copy(w0_hbm.at[q0_v], r0, sem)
    c1a = pltpu.async_copy(w1_hbm.at[q1a_v], r1a, sem)
    c1b = pltpu.async_copy(w1_hbm.at[q1b_v], r1b, sem)
    c2 = pltpu.async_copy(w2_hbm.at[idx_v], r2, sem)
    c0.wait()
    c1a.wait()
    c1b.wait()
    c2.wait()

    third = jnp.float32(_THIRD)
    hi_mask = jnp.int32(_HI_MASK)

    def expand(w):
        lo = plsc.bitcast(lax.shift_left(w, 16), jnp.float32)
        hi = plsc.bitcast(lax.bitwise_and(w, hi_mask), jnp.float32)
        return lo, hi

    def row(r, carry):
        rb = pl.multiple_of((r // _L) * _L, _L)
        vec = idx_v[pl.ds(rb, _L)]
        msk = lax.iota(jnp.int32, _L) == lax.rem(r, _L)
        i = jnp.sum(jnp.where(msk, vec, 0))
        b0 = lax.mul(lax.bitwise_and(i, 3), _W0W)
        b1 = lax.rem(lax.mul(i, _W1W), AGG)
        for p in range(AGG // (2 * _L)):
            ca = 2 * p * _L
            cb = ca + _L
            va = r2[r, pl.ds(ca, _L)]
            vb = r2[r, pl.ds(cb, _L)]
            if ca < D1:
                g = b1 + p * _L
                ga = pl.multiple_of(lax.min(g, AGG - _L), _L)
                gb = pl.multiple_of(lax.max(g - AGG, 0), _L)
                wa = r1a[r, pl.ds(ga, _L)]
                wb = r1b[r, pl.ds(gb, _L)]
                lo, hi = expand(jnp.where(g < AGG, wa, wb))
                va = va + lo
                vb = vb + hi
            if ca < D0:
                g0 = pl.multiple_of(b0 + p * _L, _L)
                lo, hi = expand(r0[r, pl.ds(g0, _L)])
                va = va + lo
                vb = vb + hi
            r2[r, pl.ds(ca, _L)] = va * third
            r2[r, pl.ds(cb, _L)] = vb * third
        return carry

    lax.fori_loop(0, _BPW, row, 0, unroll=2)

    pltpu.sync_copy(r2, out_hbm.at[pl.ds(base, _BPW)])


def _pack(W, perm, rows):
    h = jnp.take(W, jnp.asarray(perm), axis=1).astype(jnp.bfloat16)
    return lax.bitcast_convert_type(h.reshape(rows, AGG, 2), jnp.int32)


@jax.jit
def kernel(indexes, W0, W1, W2):
    idx = indexes.astype(jnp.int32)
    w0p = _pack(W0, _PERM0, _R0)
    w1p = _pack(W1, _PERM1, _R1)
    mesh = plsc.VectorSubcoreMesh(core_axis_name="c", subcore_axis_name="s")
    f = functools.partial(
        pl.kernel,
        mesh=mesh,
        out_type=jax.ShapeDtypeStruct((BATCH, AGG), jnp.float32),
        scratch_types=[
            pltpu.VMEM((_BPW,), jnp.int32),
            pltpu.VMEM((_BPW,), jnp.int32),
            pltpu.VMEM((_BPW,), jnp.int32),
            pltpu.VMEM((_BPW,), jnp.int32),
            pltpu.VMEM((_BPW, AGG), jnp.int32),
            pltpu.VMEM((_BPW, AGG), jnp.int32),
            pltpu.VMEM((_BPW, AGG), jnp.int32),
            pltpu.VMEM((_BPW, AGG), jnp.float32),
            pltpu.SemaphoreType.DMA,
        ],
        compiler_params=pltpu.CompilerParams(needs_layout_passes=False),
    )(_mean_kernel)
    return f(idx, w0p, w1p, W2)


# speedup vs baseline: 8.8573x; 8.8573x over previous
import functools

import jax
import jax.numpy as jnp
import numpy as np
from jax import lax
from jax.experimental import pallas as pl
from jax.experimental.pallas import tpu as pltpu
from jax.experimental.pallas import tpu_sc as plsc

VOCAB = 100000
D0, D1, D2 = 64, 96, 128
AGG = 128
BATCH = 4096

_R0 = VOCAB * D0 // (2 * AGG)
_R1 = VOCAB * D1 // (2 * AGG)
_W0W = D0 // 2
_W1W = D1 // 2

_info = plsc.get_sparse_core_info()
_NC, _NS, _L = _info.num_cores, _info.num_subcores, _info.num_lanes
_NW = _NC * _NS
_BPW = BATCH // _NW

_THIRD = float(np.float32(1.0) / np.float32(3.0))
_HI_MASK = np.int32(-65536)


def _interleave_perm(d):
    perm = np.empty((d,), np.int32)
    for g in range(0, d, 2 * _L):
        for k in range(_L):
            perm[g + 2 * k] = g + k
            perm[g + 2 * k + 1] = g + _L + k
    return perm

_PERM0 = _interleave_perm(D0)
_PERM1 = _interleave_perm(D1)


def _mean_kernel(idx_hbm, w0_hbm, w1_hbm, w2_hbm, out_hbm,
                 idx_v, q0_v, q1a_v, q1b_v, r0, r1a, r1b, r2, sem):
    wid = lax.axis_index("s") * _NC + lax.axis_index("c")
    base = wid * _BPW

    pltpu.sync_copy(idx_hbm.at[pl.ds(base, _BPW)], idx_v)
    for k in range(_BPW // _L):
        sl = pl.ds(k * _L, _L)
        vec = idx_v[sl]
        q0_v[sl] = lax.shift_right_logical(vec, 2)
        t = lax.shift_right_logical(vec * 3, 3)
        q1a_v[sl] = t
        q1b_v[sl] = lax.min(t + 1, jnp.full((_L,), _R1 - 1, jnp.int32))

    c0 = pltpu.async_copy(w0_hbm.at[q0_v], r0, sem)
    c1a = pltpu.async_copy(w1_hbm.at[q1a_v], r1a, sem)
    c1b = pltpu.async_copy(w1_hbm.at[q1b_v], r1b, sem)
    c2 = pltpu.async_copy(w2_hbm.at[idx_v], r2, sem)
    c0.wait()
    c1a.wait()
    c1b.wait()
    c2.wait()

    third = jnp.float32(_THIRD)
    hi_mask = jnp.int32(_HI_MASK)

    def expand(w):
        lo = plsc.bitcast(lax.shift_left(w, 16), jnp.float32)
        hi = plsc.bitcast(lax.bitwise_and(w, hi_mask), jnp.float32)
        return lo, hi

    def row(r, carry):
        rb = pl.multiple_of((r // _L) * _L, _L)
        vec = idx_v[pl.ds(rb, _L)]
        msk = lax.iota(jnp.int32, _L) == lax.rem(r, _L)
        i = jnp.sum(jnp.where(msk, vec, 0))
        b0 = lax.mul(lax.bitwise_and(i, 3), _W0W)
        b1 = lax.rem(lax.mul(i, _W1W), AGG)
        for p in range(AGG // (2 * _L)):
            ca = 2 * p * _L
            cb = ca + _L
            va = r2[r, pl.ds(ca, _L)]
            vb = r2[r, pl.ds(cb, _L)]
            if ca < D1:
                g = b1 + p * _L
                ga = pl.multiple_of(lax.min(g, AGG - _L), _L)
                gb = pl.multiple_of(lax.max(g - AGG, 0), _L)
                wa = r1a[r, pl.ds(ga, _L)]
                wb = r1b[r, pl.ds(gb, _L)]
                lo, hi = expand(jnp.where(g < AGG, wa, wb))
                va = va + lo
                vb = vb + hi
            if ca < D0:
                g0 = pl.multiple_of(b0 + p * _L, _L)
                lo, hi = expand(r0[r, pl.ds(g0, _L)])
                va = va + lo
                vb = vb + hi
            r2[r, pl.ds(ca, _L)] = va * third
            r2[r, pl.ds(cb, _L)] = vb * third
        return carry

    lax.fori_loop(0, _BPW, row, 0, unroll=2)

    pltpu.sync_copy(r2, out_hbm.at[pl.ds(base, _BPW)])


def _pack(W, rows):
    v, d = W.shape
    h = W.reshape(v, d // (2 * _L), 2, _L).swapaxes(-1, -2)
    h = h.astype(jnp.bfloat16).reshape(rows, AGG, 2)
    return lax.bitcast_convert_type(h, jnp.int32)


@jax.jit
def kernel(indexes, W0, W1, W2):
    idx = indexes.astype(jnp.int32)
    w0p = _pack(W0, _R0)
    w1p = _pack(W1, _R1)
    mesh = plsc.VectorSubcoreMesh(core_axis_name="c", subcore_axis_name="s")
    f = functools.partial(
        pl.kernel,
        mesh=mesh,
        out_type=jax.ShapeDtypeStruct((BATCH, AGG), jnp.float32),
        scratch_types=[
            pltpu.VMEM((_BPW,), jnp.int32),
            pltpu.VMEM((_BPW,), jnp.int32),
            pltpu.VMEM((_BPW,), jnp.int32),
            pltpu.VMEM((_BPW,), jnp.int32),
            pltpu.VMEM((_BPW, AGG), jnp.int32),
            pltpu.VMEM((_BPW, AGG), jnp.int32),
            pltpu.VMEM((_BPW, AGG), jnp.int32),
            pltpu.VMEM((_BPW, AGG), jnp.float32),
            pltpu.SemaphoreType.DMA,
        ],
        compiler_params=pltpu.CompilerParams(needs_layout_passes=False),
    )(_mean_kernel)
    return f(idx, w0p, w1p, W2)


# speedup vs baseline: 47.1958x; 5.3285x over previous
import functools

import jax
import jax.numpy as jnp
import numpy as np
from jax import lax
from jax.experimental import pallas as pl
from jax.experimental.pallas import tpu as pltpu
from jax.experimental.pallas import tpu_sc as plsc

VOCAB = 100000
D0, D1, D2 = 64, 96, 128
AGG = 128
BATCH = 4096

_info = plsc.get_sparse_core_info()
_NC, _NS, _L = _info.num_cores, _info.num_subcores, _info.num_lanes
_NW = _NC * _NS
_BPW = BATCH // _NW

_THIRD = float(np.float32(1.0) / np.float32(3.0))


def _mean_kernel(idx_hbm, w0_hbm, w1_hbm, w2_hbm, out_hbm,
                 idx_v, r0, r1, r2, sem, sem01):
    wid = lax.axis_index("s") * _NC + lax.axis_index("c")
    base = wid * _BPW

    pltpu.sync_copy(idx_hbm.at[pl.ds(base, _BPW)], idx_v)

    c2 = pltpu.async_copy(w2_hbm.at[idx_v], r2, sem)

    copies = []
    for k in range(_BPW // _L):
        vec = idx_v[pl.ds(k * _L, _L)]
        for j in range(_L):
            r = k * _L + j
            i0 = vec[j]
            copies.append(pltpu.async_copy(
                w0_hbm.at[pl.ds(i0, 1)], r0.at[pl.ds(r, 1)], sem01))
            copies.append(pltpu.async_copy(
                w1_hbm.at[pl.ds(i0, 1)], r1.at[pl.ds(r, 1)], sem01))
    c2.wait()
    for c in copies:
        c.wait()

    third = jnp.float32(_THIRD)

    def row(r, carry):
        for j in range(AGG // _L):
            c = j * _L
            v = r2[r, pl.ds(c, _L)]
            if c < D1:
                v = v + r1[r, pl.ds(c, _L)]
            if c < D0:
                v = v + r0[r, pl.ds(c, _L)]
            r2[r, pl.ds(c, _L)] = v * third
        return carry

    lax.fori_loop(0, _BPW, row, 0, unroll=2)

    pltpu.sync_copy(r2, out_hbm.at[pl.ds(base, _BPW)])


@jax.jit
def kernel(indexes, W0, W1, W2):
    idx = indexes.astype(jnp.int32)
    mesh = plsc.VectorSubcoreMesh(core_axis_name="c", subcore_axis_name="s")
    f = functools.partial(
        pl.kernel,
        mesh=mesh,
        out_type=jax.ShapeDtypeStruct((BATCH, AGG), jnp.float32),
        scratch_types=[
            pltpu.VMEM((_BPW,), jnp.int32),
            pltpu.VMEM((_BPW, D0), jnp.float32),
            pltpu.VMEM((_BPW, D1), jnp.float32),
            pltpu.VMEM((_BPW, D2), jnp.float32),
            pltpu.SemaphoreType.DMA,
            pltpu.SemaphoreType.DMA,
        ],
    )(_mean_kernel)
    return f(idx, W0, W1, W2)
